# Initial kernel scaffold; baseline (speedup 1.0000x reference)
import jax
import jax.numpy as jnp
from jax.experimental import pallas as pl


def kernel(x, table, W1, b1, W2, b2):
    raise NotImplementedError("write your pallas kernel here")


# speedup vs baseline: 2.5483x; 2.5483x over previous
import functools

import jax
import jax.numpy as jnp
from jax import lax
from jax.experimental import pallas as pl
from jax.experimental.pallas import tpu as pltpu
from jax.experimental.pallas import tpu_sc as plsc

B, S = 16384, 200
D = 64
H, O = 128, 64

NC, NS = 2, 16
NW = NC * NS
BPW = B // NW
CHUNK = 512
NR = CHUNK // 128
NCHUNK = BPW * S // CHUNK
XROWS_PER_CHUNK = CHUNK // 128


def _sc_body(x_hbm, table_hbm, pooled_hbm,
             idx0, idx1, rows0, rows1, dest_v, pooled_sc, sem0, sem1):
    c_id = lax.axis_index("c")
    s_id = lax.axis_index("s")
    wid = s_id * NC + c_id
    xrow_base = wid * (BPW * S // 128)
    sc_row_base = s_id * BPW

    zeros16 = jnp.zeros((16,), jnp.float32)

    def _zero_rows(i, carry):
        for k in range(D // 16):
            rows0[i, pl.ds(k * 16, 16)] = zeros16
        return carry

    lax.fori_loop(0, CHUNK, _zero_rows, 0)
    pltpu.sync_copy(rows0, pooled_sc.at[pl.ds(sc_row_base, BPW)])

    def _stage_idx(c, idx_v):
        pltpu.sync_copy(
            x_hbm.at[pl.ds(xrow_base + c * XROWS_PER_CHUNK, XROWS_PER_CHUNK)],
            idx_v)

    def _fire(idx_v, rows_v, sem):
        for r in range(NR):
            pltpu.async_copy(table_hbm.at[idx_v.at[r]],
                             rows_v.at[pl.ds(r * 128, 128)], sem)

    def _drain(idx_v, rows_v, sem):
        for r in range(NR):
            pltpu.make_async_copy(table_hbm.at[idx_v.at[r]],
                                  rows_v.at[pl.ds(r * 128, 128)], sem).wait()

    def _compute_dest(c):
        base = c * CHUNK
        for j in range(CHUNK // 16):
            q = lax.iota(jnp.int32, 16) + (base + j * 16)
            d = lax.div(q, S) + sc_row_base
            dest_v[j // 8, pl.ds((j % 8) * 16, 16)] = d

    def _scatter_add(rows_v):
        for r in range(NR):
            pltpu.sync_copy(rows_v.at[pl.ds(r * 128, 128)],
                            pooled_sc.at[dest_v.at[r]], add=True)

    _stage_idx(0, idx0)
    _fire(idx0, rows0, sem0)

    def _pair(cc, carry):
        for par in range(2):
            c = cc * 2 + par
            if par == 0:
                idx_c, rows_c, sem_c = idx0, rows0, sem0
                idx_n, rows_n, sem_n = idx1, rows1, sem1
            else:
                idx_c, rows_c, sem_c = idx1, rows1, sem1
                idx_n, rows_n, sem_n = idx0, rows0, sem0

            @pl.when(c + 1 < NCHUNK)
            def _():
                _stage_idx(c + 1, idx_n)
                _fire(idx_n, rows_n, sem_n)

            _compute_dest(c)
            _drain(idx_c, rows_c, sem_c)
            _scatter_add(rows_c)
        return carry

    lax.fori_loop(0, NCHUNK // 2, _pair, 0)

    pltpu.sync_copy(pooled_sc.at[pl.ds(sc_row_base, BPW)],
                    pooled_hbm.at[pl.ds(wid * BPW, BPW)])


_sc_pool = pl.kernel(
    _sc_body,
    out_type=jax.ShapeDtypeStruct((B, D), jnp.float32),
    mesh=plsc.VectorSubcoreMesh(core_axis_name="c", subcore_axis_name="s"),
    compiler_params=pltpu.CompilerParams(use_tc_tiling_on_sc=False),
    scratch_types=[
        pltpu.VMEM((NR, 128), jnp.int32),
        pltpu.VMEM((NR, 128), jnp.int32),
        pltpu.VMEM((CHUNK, D), jnp.float32),
        pltpu.VMEM((CHUNK, D), jnp.float32),
        pltpu.VMEM((NR, 128), jnp.int32),
        pltpu.VMEM_SHARED((NS * BPW, D), jnp.float32),
        pltpu.SemaphoreType.DMA,
        pltpu.SemaphoreType.DMA,
    ],
)


def _mlp_body(p_ref, w1_ref, b1_ref, w2_ref, b2_ref, o_ref):
    p = p_ref[...] * (1.0 / S)
    h = jnp.dot(p, w1_ref[...], preferred_element_type=jnp.float32)
    h = h + b1_ref[...]
    o = jnp.dot(h, w2_ref[...], preferred_element_type=jnp.float32)
    o_ref[...] = o + b2_ref[...]


_MLP_BLK = 1024


@functools.partial(jax.jit, static_argnums=())
def _mlp(pooled, W1, b1, W2, b2):
    return pl.pallas_call(
        _mlp_body,
        grid=(B // _MLP_BLK,),
        in_specs=[
            pl.BlockSpec((_MLP_BLK, D), lambda i: (i, 0)),
            pl.BlockSpec((D, H), lambda i: (0, 0)),
            pl.BlockSpec((1, H), lambda i: (0, 0)),
            pl.BlockSpec((H, O), lambda i: (0, 0)),
            pl.BlockSpec((1, O), lambda i: (0, 0)),
        ],
        out_specs=pl.BlockSpec((_MLP_BLK, O), lambda i: (i, 0)),
        out_shape=jax.ShapeDtypeStruct((B, O), jnp.float32),
    )(pooled, W1, b1, W2, b2)


def kernel(x, table, W1, b1, W2, b2):
    pooled = _sc_pool(x.reshape(B * S // 128, 128), table)
    return _mlp(pooled, W1, b1.reshape(1, H), W2, b2.reshape(1, O))


# speedup vs baseline: 2.5554x; 1.0028x over previous
import functools

import jax
import jax.numpy as jnp
from jax import lax
from jax.experimental import pallas as pl
from jax.experimental.pallas import tpu as pltpu
from jax.experimental.pallas import tpu_sc as plsc

B, S = 16384, 200
D = 64
H, O = 128, 64

NC, NS = 2, 16
NW = NC * NS
BPW = B // NW
CHUNK = 512
NR = CHUNK // 128
NCHUNK = BPW * S // CHUNK
XROWS_PER_CHUNK = CHUNK // 128


def _sc_body(x_hbm, table_hbm, pooled_hbm,
             idx0, idx1, rows0, rows1, dest_v, pooled_sc, sem0, sem1):
    c_id = lax.axis_index("c")
    s_id = lax.axis_index("s")
    wid = s_id * NC + c_id
    xbase = wid * (BPW * S)
    sc_row_base = s_id * BPW

    zeros16 = jnp.zeros((16,), jnp.float32)

    def _zero_rows(i, carry):
        for k in range(D // 16):
            rows0[i, pl.ds(k * 16, 16)] = zeros16
        return carry

    lax.fori_loop(0, CHUNK, _zero_rows, 0)
    pltpu.sync_copy(rows0, pooled_sc.at[pl.ds(sc_row_base, BPW)])

    def _stage_idx(c, idx_v):
        pltpu.sync_copy(x_hbm.at[pl.ds(xbase + c * CHUNK, CHUNK)], idx_v)

    def _fire(idx_v, rows_v, sem):
        pltpu.async_copy(table_hbm.at[idx_v], rows_v, sem)

    def _drain(idx_v, rows_v, sem):
        pltpu.make_async_copy(table_hbm.at[idx_v], rows_v, sem).wait()

    def _compute_dest(c):
        base = c * CHUNK
        for j in range(CHUNK // 16):
            q = lax.iota(jnp.int32, 16) + (base + j * 16)
            d = lax.div(q, S) + sc_row_base
            dest_v[pl.ds(j * 16, 16)] = d

    def _scatter_add(rows_v):
        pltpu.sync_copy(rows_v, pooled_sc.at[dest_v], add=True)

    _stage_idx(0, idx0)
    _fire(idx0, rows0, sem0)

    def _pair(cc, carry):
        for par in range(2):
            c = cc * 2 + par
            if par == 0:
                idx_c, rows_c, sem_c = idx0, rows0, sem0
                idx_n, rows_n, sem_n = idx1, rows1, sem1
            else:
                idx_c, rows_c, sem_c = idx1, rows1, sem1
                idx_n, rows_n, sem_n = idx0, rows0, sem0

            @pl.when(c + 1 < NCHUNK)
            def _():
                _stage_idx(c + 1, idx_n)
                _fire(idx_n, rows_n, sem_n)

            _compute_dest(c)
            _drain(idx_c, rows_c, sem_c)
            _scatter_add(rows_c)
        return carry

    lax.fori_loop(0, NCHUNK // 2, _pair, 0)

    pltpu.sync_copy(pooled_sc.at[pl.ds(sc_row_base, BPW)],
                    pooled_hbm.at[pl.ds(wid * BPW, BPW)])


_sc_pool = pl.kernel(
    _sc_body,
    out_type=jax.ShapeDtypeStruct((B, D), jnp.float32),
    mesh=plsc.VectorSubcoreMesh(core_axis_name="c", subcore_axis_name="s"),
    compiler_params=pltpu.CompilerParams(use_tc_tiling_on_sc=False),
    scratch_types=[
        pltpu.VMEM((CHUNK,), jnp.int32),
        pltpu.VMEM((CHUNK,), jnp.int32),
        pltpu.VMEM((CHUNK, D), jnp.float32),
        pltpu.VMEM((CHUNK, D), jnp.float32),
        pltpu.VMEM((CHUNK,), jnp.int32),
        pltpu.VMEM_SHARED((NS * BPW, D), jnp.float32),
        pltpu.SemaphoreType.DMA,
        pltpu.SemaphoreType.DMA,
    ],
)


def _mlp_body(p_ref, w1_ref, b1_ref, w2_ref, b2_ref, o_ref):
    p = p_ref[...] * (1.0 / S)
    h = jnp.dot(p, w1_ref[...], preferred_element_type=jnp.float32)
    h = h + b1_ref[...]
    o = jnp.dot(h, w2_ref[...], preferred_element_type=jnp.float32)
    o_ref[...] = o + b2_ref[...]


_MLP_BLK = 1024


@functools.partial(jax.jit, static_argnums=())
def _mlp(pooled, W1, b1, W2, b2):
    return pl.pallas_call(
        _mlp_body,
        grid=(B // _MLP_BLK,),
        in_specs=[
            pl.BlockSpec((_MLP_BLK, D), lambda i: (i, 0)),
            pl.BlockSpec((D, H), lambda i: (0, 0)),
            pl.BlockSpec((1, H), lambda i: (0, 0)),
            pl.BlockSpec((H, O), lambda i: (0, 0)),
            pl.BlockSpec((1, O), lambda i: (0, 0)),
        ],
        out_specs=pl.BlockSpec((_MLP_BLK, O), lambda i: (i, 0)),
        out_shape=jax.ShapeDtypeStruct((B, O), jnp.float32),
    )(pooled, W1, b1, W2, b2)


def kernel(x, table, W1, b1, W2, b2):
    pooled = _sc_pool(x.reshape(B * S), table)
    return _mlp(pooled, W1, b1.reshape(1, H), W2, b2.reshape(1, O))


# speedup vs baseline: 2.5579x; 1.0010x over previous
import functools

import jax
import jax.numpy as jnp
from jax import lax
from jax.experimental import pallas as pl
from jax.experimental.pallas import tpu as pltpu
from jax.experimental.pallas import tpu_sc as plsc

B, S = 16384, 200
D = 64
H, O = 128, 64

NC, NS = 2, 16
NW = NC * NS
BPW = B // NW
CHUNK = 512
NR = CHUNK // 128
NCHUNK = BPW * S // CHUNK
XROWS_PER_CHUNK = CHUNK // 128


def _sc_body(x_hbm, table_hbm, pooled_hbm,
             idx0, idx1, rows0, rows1, dest0, dest1, pooled_sc,
             sem0, sem1, ssem0, ssem1):
    c_id = lax.axis_index("c")
    s_id = lax.axis_index("s")
    wid = s_id * NC + c_id
    xbase = wid * (BPW * S)
    sc_row_base = s_id * BPW

    zeros16 = jnp.zeros((16,), jnp.float32)

    def _zero_rows(i, carry):
        for k in range(D // 16):
            rows0[i, pl.ds(k * 16, 16)] = zeros16
        return carry

    lax.fori_loop(0, CHUNK, _zero_rows, 0)
    pltpu.sync_copy(rows0, pooled_sc.at[pl.ds(sc_row_base, BPW)])

    def _stage_idx(c, idx_v):
        pltpu.sync_copy(x_hbm.at[pl.ds(xbase + c * CHUNK, CHUNK)], idx_v)

    def _fire(idx_v, rows_v, sem):
        pltpu.async_copy(table_hbm.at[idx_v], rows_v, sem)

    def _drain(idx_v, rows_v, sem):
        pltpu.make_async_copy(table_hbm.at[idx_v], rows_v, sem).wait()

    def _compute_dest(c, dest_v):
        base = c * CHUNK
        for j in range(CHUNK // 16):
            q = lax.iota(jnp.int32, 16) + (base + j * 16)
            d = lax.div(q, S) + sc_row_base
            dest_v[pl.ds(j * 16, 16)] = d

    def _fire_scatter(rows_v, dest_v, ssem):
        pltpu.async_copy(rows_v, pooled_sc.at[dest_v], ssem, add=True)

    def _drain_scatter(rows_v, dest_v, ssem):
        pltpu.make_async_copy(rows_v, pooled_sc.at[dest_v], ssem).wait()

    _stage_idx(0, idx0)
    _fire(idx0, rows0, sem0)

    def _pair(cc, carry):
        for par in range(2):
            c = cc * 2 + par
            if par == 0:
                idx_c, rows_c, sem_c, dest_c, ssem_c = idx0, rows0, sem0, dest0, ssem0
                idx_n, rows_n, sem_n, dest_n, ssem_n = idx1, rows1, sem1, dest1, ssem1
            else:
                idx_c, rows_c, sem_c, dest_c, ssem_c = idx1, rows1, sem1, dest1, ssem1
                idx_n, rows_n, sem_n, dest_n, ssem_n = idx0, rows0, sem0, dest0, ssem0

            if par == 0:
                @pl.when(cc > 0)
                def _():
                    _drain_scatter(rows_n, dest_n, ssem_n)
            else:
                _drain_scatter(rows_n, dest_n, ssem_n)

            @pl.when(c + 1 < NCHUNK)
            def _():
                _stage_idx(c + 1, idx_n)
                _fire(idx_n, rows_n, sem_n)

            _compute_dest(c, dest_c)
            _drain(idx_c, rows_c, sem_c)
            _fire_scatter(rows_c, dest_c, ssem_c)
        return carry

    lax.fori_loop(0, NCHUNK // 2, _pair, 0)

    _drain_scatter(rows1, dest1, ssem1)

    pltpu.sync_copy(pooled_sc.at[pl.ds(sc_row_base, BPW)],
                    pooled_hbm.at[pl.ds(wid * BPW, BPW)])


_sc_pool = pl.kernel(
    _sc_body,
    out_type=jax.ShapeDtypeStruct((B, D), jnp.float32),
    mesh=plsc.VectorSubcoreMesh(core_axis_name="c", subcore_axis_name="s"),
    compiler_params=pltpu.CompilerParams(use_tc_tiling_on_sc=False),
    scratch_types=[
        pltpu.VMEM((CHUNK,), jnp.int32),
        pltpu.VMEM((CHUNK,), jnp.int32),
        pltpu.VMEM((CHUNK, D), jnp.float32),
        pltpu.VMEM((CHUNK, D), jnp.float32),
        pltpu.VMEM((CHUNK,), jnp.int32),
        pltpu.VMEM((CHUNK,), jnp.int32),
        pltpu.VMEM_SHARED((NS * BPW, D), jnp.float32),
        pltpu.SemaphoreType.DMA,
        pltpu.SemaphoreType.DMA,
        pltpu.SemaphoreType.DMA,
        pltpu.SemaphoreType.DMA,
    ],
)


def _mlp_body(p_ref, w1_ref, b1_ref, w2_ref, b2_ref, o_ref):
    p = p_ref[...] * (1.0 / S)
    h = jnp.dot(p, w1_ref[...], preferred_element_type=jnp.float32)
    h = h + b1_ref[...]
    o = jnp.dot(h, w2_ref[...], preferred_element_type=jnp.float32)
    o_ref[...] = o + b2_ref[...]


_MLP_BLK = 1024


@functools.partial(jax.jit, static_argnums=())
def _mlp(pooled, W1, b1, W2, b2):
    return pl.pallas_call(
        _mlp_body,
        grid=(B // _MLP_BLK,),
        in_specs=[
            pl.BlockSpec((_MLP_BLK, D), lambda i: (i, 0)),
            pl.BlockSpec((D, H), lambda i: (0, 0)),
            pl.BlockSpec((1, H), lambda i: (0, 0)),
            pl.BlockSpec((H, O), lambda i: (0, 0)),
            pl.BlockSpec((1, O), lambda i: (0, 0)),
        ],
        out_specs=pl.BlockSpec((_MLP_BLK, O), lambda i: (i, 0)),
        out_shape=jax.ShapeDtypeStruct((B, O), jnp.float32),
    )(pooled, W1, b1, W2, b2)


def kernel(x, table, W1, b1, W2, b2):
    pooled = _sc_pool(x.reshape(B * S), table)
    return _mlp(pooled, W1, b1.reshape(1, H), W2, b2.reshape(1, O))


# speedup vs baseline: 2.5643x; 1.0025x over previous
import functools

import jax
import jax.numpy as jnp
from jax import lax
from jax.experimental import pallas as pl
from jax.experimental.pallas import tpu as pltpu
from jax.experimental.pallas import tpu_sc as plsc

B, S = 16384, 200
D = 64
H, O = 128, 64

NC, NS = 2, 16
NW = NC * NS
BPW = B // NW
NB = 4
CHUNK = 320
NCHUNK = BPW * S // CHUNK
LOOK = NB - 1


def _sc_body(x_hbm, table_hbm, pooled_hbm,
             idxs, rowss, dests, pooled_sc, gsems, ssems):
    c_id = lax.axis_index("c")
    s_id = lax.axis_index("s")
    wid = s_id * NC + c_id
    xbase = wid * (BPW * S)
    sc_row_base = s_id * BPW

    zeros16 = jnp.zeros((16,), jnp.float32)

    def _zero_rows(i, carry):
        for k in range(D // 16):
            rowss[0][i, pl.ds(k * 16, 16)] = zeros16
        return carry

    lax.fori_loop(0, CHUNK, _zero_rows, 0)
    done = 0
    while done < BPW:
        n = min(CHUNK, BPW - done)
        pltpu.sync_copy(rowss[0].at[pl.ds(0, n)],
                        pooled_sc.at[pl.ds(sc_row_base + done, n)])
        done += n

    def _stage_idx(c, idx_v):
        pltpu.sync_copy(x_hbm.at[pl.ds(xbase + c * CHUNK, CHUNK)], idx_v)

    def _fire_gather(idx_v, rows_v, sem):
        pltpu.async_copy(table_hbm.at[idx_v], rows_v, sem)

    def _drain_gather(idx_v, rows_v, sem):
        pltpu.make_async_copy(table_hbm.at[idx_v], rows_v, sem).wait()

    def _compute_dest(c, dest_v):
        base = c * CHUNK
        for j in range(CHUNK // 16):
            q = lax.iota(jnp.int32, 16) + (base + j * 16)
            d = lax.div(q, S) + sc_row_base
            dest_v[pl.ds(j * 16, 16)] = d

    def _fire_scatter(rows_v, dest_v, sem):
        pltpu.async_copy(rows_v, pooled_sc.at[dest_v], sem, add=True)

    def _drain_scatter(rows_v, dest_v, sem):
        pltpu.make_async_copy(rows_v, pooled_sc.at[dest_v], sem).wait()

    for k in range(LOOK):
        _stage_idx(k, idxs[k])
        _fire_gather(idxs[k], rowss[k], gsems[k])

    def _rev(rr, carry):
        for k in range(NB):
            c = rr * NB + k
            b = k
            fb = (k + LOOK) % NB

            if k == 0:
                @pl.when(rr > 0)
                def _():
                    _drain_scatter(rowss[fb], dests[fb], ssems[fb])
            else:
                _drain_scatter(rowss[fb], dests[fb], ssems[fb])

            @pl.when(c + LOOK < NCHUNK)
            def _():
                _stage_idx(c + LOOK, idxs[fb])
                _fire_gather(idxs[fb], rowss[fb], gsems[fb])

            _compute_dest(c, dests[b])
            _drain_gather(idxs[b], rowss[b], gsems[b])
            _fire_scatter(rowss[b], dests[b], ssems[b])
        return carry

    lax.fori_loop(0, NCHUNK // NB, _rev, 0)

    b_last = (NCHUNK - 1) % NB
    _drain_scatter(rowss[b_last], dests[b_last], ssems[b_last])

    pltpu.sync_copy(pooled_sc.at[pl.ds(sc_row_base, BPW)],
                    pooled_hbm.at[pl.ds(wid * BPW, BPW)])


_sc_pool = pl.kernel(
    _sc_body,
    out_type=jax.ShapeDtypeStruct((B, D), jnp.float32),
    mesh=plsc.VectorSubcoreMesh(core_axis_name="c", subcore_axis_name="s"),
    compiler_params=pltpu.CompilerParams(use_tc_tiling_on_sc=False),
    scratch_types=[
        [pltpu.VMEM((CHUNK,), jnp.int32) for _ in range(NB)],
        [pltpu.VMEM((CHUNK, D), jnp.float32) for _ in range(NB)],
        [pltpu.VMEM((CHUNK,), jnp.int32) for _ in range(NB)],
        pltpu.VMEM_SHARED((NS * BPW, D), jnp.float32),
        [pltpu.SemaphoreType.DMA for _ in range(NB)],
        [pltpu.SemaphoreType.DMA for _ in range(NB)],
    ],
)


def _mlp_body(p_ref, w1_ref, b1_ref, w2_ref, b2_ref, o_ref):
    p = p_ref[...] * (1.0 / S)
    h = jnp.dot(p, w1_ref[...], preferred_element_type=jnp.float32)
    h = h + b1_ref[...]
    o = jnp.dot(h, w2_ref[...], preferred_element_type=jnp.float32)
    o_ref[...] = o + b2_ref[...]


_MLP_BLK = 1024


@functools.partial(jax.jit, static_argnums=())
def _mlp(pooled, W1, b1, W2, b2):
    return pl.pallas_call(
        _mlp_body,
        grid=(B // _MLP_BLK,),
        in_specs=[
            pl.BlockSpec((_MLP_BLK, D), lambda i: (i, 0)),
            pl.BlockSpec((D, H), lambda i: (0, 0)),
            pl.BlockSpec((1, H), lambda i: (0, 0)),
            pl.BlockSpec((H, O), lambda i: (0, 0)),
            pl.BlockSpec((1, O), lambda i: (0, 0)),
        ],
        out_specs=pl.BlockSpec((_MLP_BLK, O), lambda i: (i, 0)),
        out_shape=jax.ShapeDtypeStruct((B, O), jnp.float32),
    )(pooled, W1, b1, W2, b2)


def kernel(x, table, W1, b1, W2, b2):
    pooled = _sc_pool(x.reshape(B * S), table)
    return _mlp(pooled, W1, b1.reshape(1, H), W2, b2.reshape(1, O))


# speedup vs baseline: 2.7265x; 1.0633x over previous
import functools

import jax
import jax.numpy as jnp
from jax import lax
from jax.experimental import pallas as pl
from jax.experimental.pallas import tpu as pltpu
from jax.experimental.pallas import tpu_sc as plsc

B, S = 16384, 200
D = 64
H, O = 128, 64

NC, NS = 2, 16
NW = NC * NS
BPW = B // NW
NB = 4
CHUNK = S
NCHUNK = BPW
LOOK = NB - 1
UNROLL = 4


def _sc_body(x_hbm, table_hbm, pooled_hbm, idxs, rowss, pooled_v, gsems):
    c_id = lax.axis_index("c")
    s_id = lax.axis_index("s")
    wid = s_id * NC + c_id
    xbase = wid * (BPW * S)

    def _stage_idx(c, idx_v):
        pltpu.sync_copy(x_hbm.at[pl.ds(xbase + c * CHUNK, CHUNK)], idx_v)

    def _fire_gather(idx_v, rows_v, sem):
        pltpu.async_copy(table_hbm.at[idx_v], rows_v, sem)

    def _drain_gather(idx_v, rows_v, sem):
        pltpu.make_async_copy(table_hbm.at[idx_v], rows_v, sem).wait()

    zero16 = jnp.zeros((16,), jnp.float32)

    def _accumulate(c, rows_v):
        def step(t, accs):
            accs = list(accs)
            for u in range(UNROLL):
                i = t * UNROLL + u
                for k in range(D // 16):
                    accs[(u % 2) * 4 + k] = accs[(u % 2) * 4 + k] + \
                        rows_v[i, pl.ds(k * 16, 16)]
            return tuple(accs)

        accs = lax.fori_loop(0, CHUNK // UNROLL, step,
                             tuple(zero16 for _ in range(8)))
        for k in range(D // 16):
            pooled_v[c, pl.ds(k * 16, 16)] = accs[k] + accs[4 + k]

    for k in range(LOOK):
        _stage_idx(k, idxs[k])
        _fire_gather(idxs[k], rowss[k], gsems[k])

    def _rev(rr, carry):
        for k in range(NB):
            c = rr * NB + k
            b = k
            fb = (k + LOOK) % NB

            @pl.when(c + LOOK < NCHUNK)
            def _():
                _stage_idx(c + LOOK, idxs[fb])
                _fire_gather(idxs[fb], rowss[fb], gsems[fb])

            _drain_gather(idxs[b], rowss[b], gsems[b])
            _accumulate(c, rowss[b])
        return carry

    lax.fori_loop(0, NCHUNK // NB, _rev, 0)

    pltpu.sync_copy(pooled_v, pooled_hbm.at[pl.ds(wid * BPW, BPW)])


_sc_pool = pl.kernel(
    _sc_body,
    out_type=jax.ShapeDtypeStruct((B, D), jnp.float32),
    mesh=plsc.VectorSubcoreMesh(core_axis_name="c", subcore_axis_name="s"),
    compiler_params=pltpu.CompilerParams(use_tc_tiling_on_sc=False),
    scratch_types=[
        [pltpu.VMEM((CHUNK,), jnp.int32) for _ in range(NB)],
        [pltpu.VMEM((CHUNK, D), jnp.float32) for _ in range(NB)],
        pltpu.VMEM((BPW, D), jnp.float32),
        [pltpu.SemaphoreType.DMA for _ in range(NB)],
    ],
)


def _mlp_body(p_ref, w1_ref, b1_ref, w2_ref, b2_ref, o_ref):
    p = p_ref[...] * (1.0 / S)
    h = jnp.dot(p, w1_ref[...], preferred_element_type=jnp.float32)
    h = h + b1_ref[...]
    o = jnp.dot(h, w2_ref[...], preferred_element_type=jnp.float32)
    o_ref[...] = o + b2_ref[...]


_MLP_BLK = 1024


@functools.partial(jax.jit, static_argnums=())
def _mlp(pooled, W1, b1, W2, b2):
    return pl.pallas_call(
        _mlp_body,
        grid=(B // _MLP_BLK,),
        in_specs=[
            pl.BlockSpec((_MLP_BLK, D), lambda i: (i, 0)),
            pl.BlockSpec((D, H), lambda i: (0, 0)),
            pl.BlockSpec((1, H), lambda i: (0, 0)),
            pl.BlockSpec((H, O), lambda i: (0, 0)),
            pl.BlockSpec((1, O), lambda i: (0, 0)),
        ],
        out_specs=pl.BlockSpec((_MLP_BLK, O), lambda i: (i, 0)),
        out_shape=jax.ShapeDtypeStruct((B, O), jnp.float32),
    )(pooled, W1, b1, W2, b2)


def kernel(x, table, W1, b1, W2, b2):
    pooled = _sc_pool(x.reshape(B * S), table)
    return _mlp(pooled, W1, b1.reshape(1, H), W2, b2.reshape(1, O))
